# baseline (speedup 1.0000x reference)
import functools

import jax
import jax.numpy as jnp
from jax import lax
from jax.experimental import pallas as pl
from jax.experimental.pallas import tpu as pltpu
from jax.experimental.pallas import tpu_sc as plsc

B = 128
V = 50000
S = 512
FEAT = 2560
NC = 2
NS = 16
L = 16
NU = 64
CH = 784
CH_TAIL = V - 63 * CH
NUPD = B * S
BLK = 8192
ROWS_PER_TILE = B // NS
BIG = 1 << 29

_MESH = plsc.VectorSubcoreMesh(
    core_axis_name="c", subcore_axis_name="s", num_cores=NC, num_subcores=NS
)


@functools.partial(
    pl.kernel,
    out_type=jax.ShapeDtypeStruct((V, B), jnp.float32),
    mesh=_MESH,
    compiler_params=pltpu.CompilerParams(needs_layout_passes=False),
    scratch_types=[
        pltpu.VMEM((CH, B), jnp.float32),
        pltpu.VMEM((BLK,), jnp.int32),
        pltpu.VMEM((BLK,), jnp.float32),
        pltpu.VMEM((FEAT,), jnp.float32),
        pltpu.VMEM((FEAT,), jnp.float32),
        pltpu.VMEM((L,), jnp.float32),
        pltpu.VMEM((L,), jnp.float32),
        pltpu.VMEM((L,), jnp.float32),
        pltpu.VMEM((NS * L,), jnp.float32),
        pltpu.VMEM((B,), jnp.float32),
        pltpu.VMEM((B,), jnp.float32),
        pltpu.VMEM_SHARED((NS * L,), jnp.float32),
        pltpu.SemaphoreType.DMA,
        pltpu.SemaphoreType.DMA,
    ],
)
def _pg_kernel(vt_hbm, feat_hbm, cell_hbm, val_hbm, w_hbm, b_hbm,
               out_hbm, ubuf, cell_v, upd_v, feat_v, w_v, b_v, red_v,
               pub_v, pgl_v, pg_v, om_v, shared, sem_in, sem_out):
    cid = lax.axis_index("c")
    sid = lax.axis_index("s")
    wid = sid * NC + cid
    lo0 = pl.multiple_of(wid * CH, 8)

    in_desc = pltpu.async_copy(vt_hbm.at[pl.ds(lo0, CH)], ubuf, sem_in)

    pltpu.sync_copy(w_hbm, w_v)
    pltpu.sync_copy(b_hbm, b_v)

    lanes = lax.iota(jnp.int32, L)
    pub = jnp.zeros((L,), jnp.float32)
    for r in range(ROWS_PER_TILE):
        row = sid * ROWS_PER_TILE + r
        pltpu.sync_copy(feat_hbm.at[row], feat_v)

        def dot_body(i, a):
            return a + feat_v[pl.ds(i, L)] * w_v[pl.ds(i, L)]

        acc = plsc.parallel_loop(0, FEAT, L, carry=b_v[:])(dot_body)
        red_v[:] = plsc.cumsum(acc)
        tot = plsc.load_gather(red_v, [jnp.full((L,), L - 1, jnp.int32)])
        pg = 1.0 / (1.0 + jnp.exp(-tot))
        pub = jnp.where(lanes == r, pg, pub)
        pub = jnp.where(lanes == r + ROWS_PER_TILE, 1.0 - pg, pub)
    pub_v[:] = pub
    pltpu.sync_copy(pub_v, shared.at[pl.ds(pl.multiple_of(sid * L, L), L)])
    plsc.subcore_barrier()
    pltpu.sync_copy(shared, pgl_v)

    for c in range(B // L):
        g = jnp.broadcast_to(jnp.int32(c * L), (L,)) + lanes
        gslot = (g // ROWS_PER_TILE) * L + lax.rem(g, ROWS_PER_TILE)
        pg_v[pl.ds(c * L, L)] = plsc.load_gather(pgl_v, [gslot])
        om_v[pl.ds(c * L, L)] = plsc.load_gather(
            pgl_v, [gslot + ROWS_PER_TILE])

    pgc = [pg_v[pl.ds(c * L, L)] for c in range(B // L)]

    out_desc = None
    for u in range(2):
        uid = wid + 32 * u
        lo = pl.multiple_of(uid * CH, 8)
        if u == 1:
            is_tail = uid == NU - 1
            out_desc.wait()

            @pl.when(is_tail)
            def _():
                pltpu.sync_copy(vt_hbm.at[pl.ds(lo, CH_TAIL)],
                                ubuf.at[pl.ds(0, CH_TAIL)])

            @pl.when(jnp.logical_not(is_tail))
            def _():
                pltpu.sync_copy(vt_hbm.at[pl.ds(lo, CH)], ubuf)

            cnt = jnp.where(is_tail, CH_TAIL, CH)
        else:
            in_desc.wait()
            cnt = jnp.int32(CH)

        def scale_body(i):
            for c in range(B // L):
                ubuf[i, pl.ds(c * L, L)] = ubuf[i, pl.ds(c * L, L)] * pgc[c]

        plsc.parallel_loop(0, CH_TAIL, 1, unroll=2)(scale_body)
        if u == 0:
            plsc.parallel_loop(CH_TAIL, CH, 1, unroll=2)(scale_body)
        else:
            @pl.when(jnp.logical_not(is_tail))
            def _():
                plsc.parallel_loop(CH_TAIL, CH, 1, unroll=2)(scale_body)

        clo = jnp.broadcast_to(lo * B, (L,))
        chi = jnp.broadcast_to((lo + cnt) * B, (L,))
        for blk in range(NUPD // BLK):
            pltpu.sync_copy(cell_hbm.at[pl.ds(blk * BLK, BLK)], cell_v)
            pltpu.sync_copy(val_hbm.at[pl.ds(blk * BLK, BLK)], upd_v)

            def scan_body(i):
                t = cell_v[pl.ds(i, L)]
                a = upd_v[pl.ds(i, L)]
                m = (t >= clo) & (t < chi)
                tl = jnp.minimum(jnp.maximum(t - clo, 0), CH * B - 1)
                bb = t & jnp.int32(B - 1)
                om = plsc.load_gather(om_v, [bb])
                plsc.addupdate_scatter(
                    ubuf, [tl >> 7, tl & jnp.int32(B - 1)], a * om, mask=m)

            plsc.parallel_loop(0, BLK, L)(scan_body)

        if u == 1:
            @pl.when(is_tail)
            def _():
                pltpu.sync_copy(ubuf.at[pl.ds(0, CH_TAIL)],
                                out_hbm.at[pl.ds(lo, CH_TAIL)])

            @pl.when(jnp.logical_not(is_tail))
            def _():
                pltpu.sync_copy(ubuf, out_hbm.at[pl.ds(lo, CH)])
        else:
            out_desc = pltpu.async_copy(ubuf, out_hbm.at[pl.ds(lo, CH)],
                                        sem_out)


def kernel(vocab_dist, attn_dist, context, state, emb, src_ids, vocab_size,
           W_c, W_s, W_y, b):
    feat = jnp.concatenate([context, state, emb], axis=1)
    w = jnp.concatenate([W_c[0], W_s[0], W_y[0]])
    b16 = jnp.pad(b.astype(jnp.float32), (0, L - 1))
    src = src_ids.astype(jnp.int32)
    valid = src < vocab_size
    cells = jnp.where(valid, src * B + jnp.arange(B, dtype=jnp.int32)[:, None],
                      jnp.int32(BIG)).reshape(-1)
    vals = attn_dist.reshape(-1)
    vt = vocab_dist.T
    out_t = _pg_kernel(vt, feat, cells, vals, w, b16)
    return out_t.T


# speedup vs baseline: 1.2601x; 1.2601x over previous
import functools

import jax
import jax.numpy as jnp
from jax import lax
from jax.experimental import pallas as pl
from jax.experimental.pallas import tpu as pltpu
from jax.experimental.pallas import tpu_sc as plsc

B = 128
V = 50000
S = 512
FEAT = 2560
NC = 2
NS = 16
L = 16
NU = 64
CH = 784
CH_TAIL = V - 63 * CH
NUPD = B * S
BLK = 4096
NBLK = NUPD // BLK
ROWS_PER_TILE = B // NS
BIG = 1 << 29

_MESH = plsc.VectorSubcoreMesh(
    core_axis_name="c", subcore_axis_name="s", num_cores=NC, num_subcores=NS
)


@functools.partial(
    pl.kernel,
    out_type=jax.ShapeDtypeStruct((V, B), jnp.float32),
    mesh=_MESH,
    compiler_params=pltpu.CompilerParams(needs_layout_passes=False),
    scratch_types=[
        pltpu.VMEM((CH, B), jnp.float32),
        pltpu.VMEM((2, BLK), jnp.int32),
        pltpu.VMEM((2, BLK), jnp.float32),
        pltpu.VMEM((2, FEAT), jnp.float32),
        pltpu.VMEM((FEAT,), jnp.float32),
        pltpu.VMEM((L,), jnp.float32),
        pltpu.VMEM((L,), jnp.float32),
        pltpu.VMEM((L,), jnp.float32),
        pltpu.VMEM((NS * L,), jnp.float32),
        pltpu.VMEM((B,), jnp.float32),
        pltpu.VMEM((B,), jnp.float32),
        pltpu.VMEM_SHARED((NS * L,), jnp.float32),
        pltpu.SemaphoreType.DMA,
        pltpu.SemaphoreType.DMA,
        pltpu.SemaphoreType.DMA,
        pltpu.SemaphoreType.DMA,
        pltpu.SemaphoreType.DMA,
    ],
)
def _pg_kernel(vt_hbm, feat_hbm, cell_hbm, val_hbm, w_hbm, b_hbm,
               out_hbm, ubuf, cell_v, upd_v, feat_v, w_v, b_v, red_v,
               pub_v, pgl_v, pg_v, om_v, shared, sem_in, sem_out,
               sem_u0, sem_u1, sem_f):
    cid = lax.axis_index("c")
    sid = lax.axis_index("s")
    wid = sid * NC + cid
    lo0 = pl.multiple_of(wid * CH, 8)
    sem_u = [sem_u0, sem_u1]

    in_desc = pltpu.async_copy(vt_hbm.at[pl.ds(lo0, CH)], ubuf, sem_in)

    pltpu.sync_copy(w_hbm, w_v)
    pltpu.sync_copy(b_hbm, b_v)

    lanes = lax.iota(jnp.int32, L)
    pub = jnp.zeros((L,), jnp.float32)
    fdesc = [None, None]
    fdesc[0] = pltpu.async_copy(
        feat_hbm.at[sid * ROWS_PER_TILE], feat_v.at[0], sem_f)
    for r in range(ROWS_PER_TILE):
        if r + 1 < ROWS_PER_TILE:
            fdesc[(r + 1) % 2] = pltpu.async_copy(
                feat_hbm.at[sid * ROWS_PER_TILE + r + 1],
                feat_v.at[(r + 1) % 2], sem_f)
        fdesc[r % 2].wait()
        fr = r % 2

        def dot_body(i, a):
            return a + feat_v[fr, pl.ds(i, L)] * w_v[pl.ds(i, L)]

        acc = plsc.parallel_loop(0, FEAT, L, unroll=4, carry=b_v[:])(dot_body)
        red_v[:] = plsc.cumsum(acc)
        tot = plsc.load_gather(red_v, [jnp.full((L,), L - 1, jnp.int32)])
        pg = 1.0 / (1.0 + jnp.exp(-tot))
        pub = jnp.where(lanes == r, pg, pub)
        pub = jnp.where(lanes == r + ROWS_PER_TILE, 1.0 - pg, pub)
    pub_v[:] = pub
    pltpu.sync_copy(pub_v, shared.at[pl.ds(pl.multiple_of(sid * L, L), L)])
    plsc.subcore_barrier()
    pltpu.sync_copy(shared, pgl_v)

    for c in range(B // L):
        g = jnp.broadcast_to(jnp.int32(c * L), (L,)) + lanes
        gslot = (g // ROWS_PER_TILE) * L + lax.rem(g, ROWS_PER_TILE)
        pg_v[pl.ds(c * L, L)] = plsc.load_gather(pgl_v, [gslot])
        om_v[pl.ds(c * L, L)] = plsc.load_gather(
            pgl_v, [gslot + ROWS_PER_TILE])

    pgc = [pg_v[pl.ds(c * L, L)] for c in range(B // L)]

    out_desc = None
    for u in range(2):
        uid = wid + 32 * u
        lo = pl.multiple_of(uid * CH, 8)
        if u == 1:
            is_tail = uid == NU - 1
            out_desc.wait()

            @pl.when(is_tail)
            def _():
                pltpu.sync_copy(vt_hbm.at[pl.ds(lo, CH_TAIL)],
                                ubuf.at[pl.ds(0, CH_TAIL)])

            @pl.when(jnp.logical_not(is_tail))
            def _():
                pltpu.sync_copy(vt_hbm.at[pl.ds(lo, CH)], ubuf)

            cnt = jnp.where(is_tail, CH_TAIL, CH)
        else:
            in_desc.wait()
            cnt = jnp.int32(CH)

        def scale_body(i):
            for c in range(B // L):
                ubuf[i, pl.ds(c * L, L)] = ubuf[i, pl.ds(c * L, L)] * pgc[c]

        plsc.parallel_loop(0, CH_TAIL, 1, unroll=2)(scale_body)
        if u == 0:
            plsc.parallel_loop(CH_TAIL, CH, 1, unroll=2)(scale_body)
        else:
            @pl.when(jnp.logical_not(is_tail))
            def _():
                plsc.parallel_loop(CH_TAIL, CH, 1, unroll=2)(scale_body)

        clo = jnp.broadcast_to(lo * B, (L,))
        chi = jnp.broadcast_to((lo + cnt) * B, (L,))
        udescs = [None] * NBLK
        udescs[0] = (
            pltpu.async_copy(cell_hbm.at[pl.ds(0, BLK)], cell_v.at[0],
                             sem_u[0]),
            pltpu.async_copy(val_hbm.at[pl.ds(0, BLK)], upd_v.at[0],
                             sem_u[0]),
        )
        for blk in range(NBLK):
            slot = blk % 2
            if blk + 1 < NBLK:
                nslot = (blk + 1) % 2
                udescs[blk + 1] = (
                    pltpu.async_copy(
                        cell_hbm.at[pl.ds((blk + 1) * BLK, BLK)],
                        cell_v.at[nslot], sem_u[nslot]),
                    pltpu.async_copy(
                        val_hbm.at[pl.ds((blk + 1) * BLK, BLK)],
                        upd_v.at[nslot], sem_u[nslot]),
                )
            udescs[blk][0].wait()
            udescs[blk][1].wait()

            def scan_body(i):
                t = cell_v[slot, pl.ds(i, L)]
                a = upd_v[slot, pl.ds(i, L)]
                m = (t >= clo) & (t < chi)
                tl = jnp.minimum(jnp.maximum(t - clo, 0), CH * B - 1)
                bb = t & jnp.int32(B - 1)
                om = plsc.load_gather(om_v, [bb])
                plsc.addupdate_scatter(
                    ubuf, [tl >> 7, tl & jnp.int32(B - 1)], a * om, mask=m)

            plsc.parallel_loop(0, BLK, L)(scan_body)

        if u == 1:
            @pl.when(is_tail)
            def _():
                pltpu.sync_copy(ubuf.at[pl.ds(0, CH_TAIL)],
                                out_hbm.at[pl.ds(lo, CH_TAIL)])

            @pl.when(jnp.logical_not(is_tail))
            def _():
                pltpu.sync_copy(ubuf, out_hbm.at[pl.ds(lo, CH)])
        else:
            out_desc = pltpu.async_copy(ubuf, out_hbm.at[pl.ds(lo, CH)],
                                        sem_out)


def kernel(vocab_dist, attn_dist, context, state, emb, src_ids, vocab_size,
           W_c, W_s, W_y, b):
    feat = jnp.concatenate([context, state, emb], axis=1)
    w = jnp.concatenate([W_c[0], W_s[0], W_y[0]])
    b16 = jnp.pad(b.astype(jnp.float32), (0, L - 1))
    src = src_ids.astype(jnp.int32)
    valid = src < vocab_size
    cells = jnp.where(valid,
                      src * B + jnp.arange(B, dtype=jnp.int32)[:, None],
                      jnp.int32(BIG)).reshape(-1)
    vals = attn_dist.reshape(-1)
    vt = vocab_dist.T
    out_t = _pg_kernel(vt, feat, cells, vals, w, b16)
    return out_t.T


# speedup vs baseline: 1.3885x; 1.1019x over previous
import functools

import jax
import jax.numpy as jnp
from jax import lax
from jax.experimental import pallas as pl
from jax.experimental.pallas import tpu as pltpu
from jax.experimental.pallas import tpu_sc as plsc

B = 128
V = 50000
S = 512
FEAT = 2560
NC = 2
NS = 16
L = 16
NU = 64
CH = 784
CH_TAIL = V - 63 * CH
NUPD = B * S
BLK = 4096
NBLK = NUPD // BLK
ROWS_PER_TILE = B // NS
BIG = 1 << 29

_MESH = plsc.VectorSubcoreMesh(
    core_axis_name="c", subcore_axis_name="s", num_cores=NC, num_subcores=NS
)


@functools.partial(
    pl.kernel,
    out_type=jax.ShapeDtypeStruct((V, B), jnp.float32),
    mesh=_MESH,
    compiler_params=pltpu.CompilerParams(needs_layout_passes=False),
    scratch_types=[
        pltpu.VMEM((CH, B), jnp.float32),
        pltpu.VMEM((2, BLK), jnp.int32),
        pltpu.VMEM((2, BLK), jnp.float32),
        pltpu.VMEM((2, FEAT), jnp.float32),
        pltpu.VMEM((FEAT,), jnp.float32),
        pltpu.VMEM((L,), jnp.float32),
        pltpu.VMEM((L,), jnp.float32),
        pltpu.VMEM((L,), jnp.float32),
        pltpu.VMEM((NS * L,), jnp.float32),
        pltpu.VMEM((B,), jnp.float32),
        pltpu.VMEM((B,), jnp.float32),
        pltpu.VMEM_SHARED((NS * L,), jnp.float32),
        pltpu.SemaphoreType.DMA,
        pltpu.SemaphoreType.DMA,
        pltpu.SemaphoreType.DMA,
        pltpu.SemaphoreType.DMA,
        pltpu.SemaphoreType.DMA,
    ],
)
def _pg_kernel(vt_hbm, feat_hbm, cell_hbm, val_hbm, w_hbm, b_hbm,
               out_hbm, ubuf, cell_v, upd_v, feat_v, w_v, b_v, red_v,
               pub_v, pgl_v, pg_v, om_v, shared, sem_in, sem_out,
               sem_u0, sem_u1, sem_f):
    cid = lax.axis_index("c")
    sid = lax.axis_index("s")
    wid = sid * NC + cid
    lo0 = pl.multiple_of(wid * CH, 8)
    sem_u = [sem_u0, sem_u1]

    in_desc = pltpu.async_copy(vt_hbm.at[pl.ds(lo0, CH)], ubuf, sem_in)

    pltpu.sync_copy(w_hbm, w_v)
    pltpu.sync_copy(b_hbm, b_v)

    lanes = lax.iota(jnp.int32, L)
    pub = jnp.zeros((L,), jnp.float32)
    fdesc = [None, None]
    fdesc[0] = pltpu.async_copy(
        feat_hbm.at[sid * ROWS_PER_TILE], feat_v.at[0], sem_f)
    for r in range(ROWS_PER_TILE):
        if r + 1 < ROWS_PER_TILE:
            fdesc[(r + 1) % 2] = pltpu.async_copy(
                feat_hbm.at[sid * ROWS_PER_TILE + r + 1],
                feat_v.at[(r + 1) % 2], sem_f)
        fdesc[r % 2].wait()
        fr = r % 2

        def dot_body(i, a):
            return a + feat_v[fr, pl.ds(i, L)] * w_v[pl.ds(i, L)]

        acc = plsc.parallel_loop(0, FEAT, L, unroll=4, carry=b_v[:])(dot_body)
        red_v[:] = plsc.cumsum(acc)
        tot = plsc.load_gather(red_v, [jnp.full((L,), L - 1, jnp.int32)])
        pg = 1.0 / (1.0 + jnp.exp(-tot))
        pub = jnp.where(lanes == r, pg, pub)
        pub = jnp.where(lanes == r + ROWS_PER_TILE, 1.0 - pg, pub)
    pub_v[:] = pub
    pltpu.sync_copy(pub_v, shared.at[pl.ds(pl.multiple_of(sid * L, L), L)])
    plsc.subcore_barrier()
    pltpu.sync_copy(shared, pgl_v)

    for c in range(B // L):
        g = jnp.broadcast_to(jnp.int32(c * L), (L,)) + lanes
        gslot = (g // ROWS_PER_TILE) * L + lax.rem(g, ROWS_PER_TILE)
        pg_v[pl.ds(c * L, L)] = plsc.load_gather(pgl_v, [gslot])
        om_v[pl.ds(c * L, L)] = plsc.load_gather(
            pgl_v, [gslot + ROWS_PER_TILE])

    pgc = [pg_v[pl.ds(c * L, L)] for c in range(B // L)]

    out_desc = None
    for u in range(2):
        uid = wid + 32 * u
        lo = pl.multiple_of(uid * CH, 8)
        if u == 1:
            is_tail = uid == NU - 1
            out_desc.wait()

            @pl.when(is_tail)
            def _():
                pltpu.sync_copy(vt_hbm.at[pl.ds(lo, CH_TAIL)],
                                ubuf.at[pl.ds(0, CH_TAIL)])

            @pl.when(jnp.logical_not(is_tail))
            def _():
                pltpu.sync_copy(vt_hbm.at[pl.ds(lo, CH)], ubuf)

            cnt = jnp.where(is_tail, CH_TAIL, CH)
        else:
            in_desc.wait()
            cnt = jnp.int32(CH)

        def scale_body(i):
            for c in range(B // L):
                ubuf[i, pl.ds(c * L, L)] = ubuf[i, pl.ds(c * L, L)] * pgc[c]

        plsc.parallel_loop(0, CH_TAIL, 1, unroll=2)(scale_body)
        if u == 0:
            plsc.parallel_loop(CH_TAIL, CH, 1, unroll=2)(scale_body)
        else:
            @pl.when(jnp.logical_not(is_tail))
            def _():
                plsc.parallel_loop(CH_TAIL, CH, 1, unroll=2)(scale_body)

        clo = jnp.broadcast_to(lo * B, (L,))
        chi = jnp.broadcast_to((lo + cnt) * B, (L,))
        udescs = [None] * NBLK
        udescs[0] = (
            pltpu.async_copy(cell_hbm.at[pl.ds(0, BLK)], cell_v.at[0],
                             sem_u[0]),
            pltpu.async_copy(val_hbm.at[pl.ds(0, BLK)], upd_v.at[0],
                             sem_u[0]),
        )
        for blk in range(NBLK):
            slot = blk % 2
            if blk + 1 < NBLK:
                nslot = (blk + 1) % 2
                udescs[blk + 1] = (
                    pltpu.async_copy(
                        cell_hbm.at[pl.ds((blk + 1) * BLK, BLK)],
                        cell_v.at[nslot], sem_u[nslot]),
                    pltpu.async_copy(
                        val_hbm.at[pl.ds((blk + 1) * BLK, BLK)],
                        upd_v.at[nslot], sem_u[nslot]),
                )
            udescs[blk][0].wait()
            udescs[blk][1].wait()

            def scan_body(i):
                t = cell_v[slot, pl.ds(i, L)]
                a = upd_v[slot, pl.ds(i, L)]
                m = (t >= clo) & (t < chi)
                tl = jnp.minimum(jnp.maximum(t - clo, 0), CH * B - 1)
                bb = t & jnp.int32(B - 1)
                om = plsc.load_gather(om_v, [bb])
                plsc.addupdate_scatter(
                    ubuf, [tl >> 7, tl & jnp.int32(B - 1)], a * om, mask=m)

            plsc.parallel_loop(0, BLK, L, unroll=4)(scan_body)

        if u == 1:
            @pl.when(is_tail)
            def _():
                pltpu.sync_copy(ubuf.at[pl.ds(0, CH_TAIL)],
                                out_hbm.at[pl.ds(lo, CH_TAIL)])

            @pl.when(jnp.logical_not(is_tail))
            def _():
                pltpu.sync_copy(ubuf, out_hbm.at[pl.ds(lo, CH)])
        else:
            out_desc = pltpu.async_copy(ubuf, out_hbm.at[pl.ds(lo, CH)],
                                        sem_out)


def kernel(vocab_dist, attn_dist, context, state, emb, src_ids, vocab_size,
           W_c, W_s, W_y, b):
    feat = jnp.concatenate([context, state, emb], axis=1)
    w = jnp.concatenate([W_c[0], W_s[0], W_y[0]])
    b16 = jnp.pad(b.astype(jnp.float32), (0, L - 1))
    src = src_ids.astype(jnp.int32)
    valid = src < vocab_size
    cells = jnp.where(valid,
                      src * B + jnp.arange(B, dtype=jnp.int32)[:, None],
                      jnp.int32(BIG)).reshape(-1)
    vals = attn_dist.reshape(-1)
    vt = vocab_dist.T
    out_t = _pg_kernel(vt, feat, cells, vals, w, b16)
    return out_t.T
